# Initial kernel scaffold; baseline (speedup 1.0000x reference)
import jax
import jax.numpy as jnp
from jax.experimental import pallas as pl


def kernel(x, edge_index, edge_weight, W1, b1, W2, b2):
    raise NotImplementedError("write your pallas kernel here")


# speedup vs baseline: 3.9832x; 3.9832x over previous
import functools

import jax
import jax.numpy as jnp
from jax import lax
from jax.experimental import pallas as pl
from jax.experimental.pallas import tpu as pltpu, tpu_sc as plsc

N = 10000
NACC = 10240
E = 320000
D = 128
SUB = 16
CORES = 2
CH = 128
NCH = 79
EPS = NCH * CH
EPAD = CORES * SUB * EPS
RPT = NACC // SUB


def _mm1_body(x_ref, w_ref, b_ref, o_ref):
    o_ref[...] = (
        jnp.dot(x_ref[...], w_ref[...], preferred_element_type=jnp.float32)
        + b_ref[...]
    )


def _matmul1(x, W, b):
    BM = 400
    return pl.pallas_call(
        _mm1_body,
        grid=(N // BM,),
        in_specs=[
            pl.BlockSpec((BM, D), lambda i: (i, 0)),
            pl.BlockSpec((D, D), lambda i: (0, 0)),
            pl.BlockSpec((1, D), lambda i: (0, 0)),
        ],
        out_specs=pl.BlockSpec((BM, D), lambda i: (i, 0)),
        out_shape=jax.ShapeDtypeStruct((N, D), jnp.float32),
    )(x, W, b)


def _mm2_body(a_ref, b_ref, w_ref, bias_ref, o_ref):
    x = jnp.maximum(a_ref[0] + b_ref[0], 0.0)
    o_ref[...] = (
        jnp.dot(x, w_ref[...], preferred_element_type=jnp.float32)
        + bias_ref[...]
    )


def _matmul2(parts, W, b):
    BM = 400
    return pl.pallas_call(
        _mm2_body,
        grid=(N // BM,),
        in_specs=[
            pl.BlockSpec((1, BM, D), lambda i: (0, i, 0)),
            pl.BlockSpec((1, BM, D), lambda i: (1, i, 0)),
            pl.BlockSpec((D, D), lambda i: (0, 0)),
            pl.BlockSpec((1, D), lambda i: (0, 0)),
        ],
        out_specs=pl.BlockSpec((BM, D), lambda i: (i, 0)),
        out_shape=jax.ShapeDtypeStruct((N, D), jnp.float32),
    )(parts, parts, W, b)


def _add_body(a_ref, b_ref, o_ref):
    o_ref[...] = a_ref[0] + b_ref[0]


def _add_parts(parts):
    BM = 400
    return pl.pallas_call(
        _add_body,
        grid=(N // BM,),
        in_specs=[
            pl.BlockSpec((1, BM, D), lambda i: (0, i, 0)),
            pl.BlockSpec((1, BM, D), lambda i: (1, i, 0)),
        ],
        out_specs=pl.BlockSpec((BM, D), lambda i: (i, 0)),
        out_shape=jax.ShapeDtypeStruct((N, D), jnp.float32),
    )(parts, parts)


def _conv_body(h_hbm, src_hbm, dst_hbm, w_hbm, out_hbm,
               src_v, dst_v, w_v, rows_v, accum, sem):
    c = lax.axis_index("c")
    s = lax.axis_index("s")

    pltpu.sync_copy(src_hbm.at[c, s], src_v)
    pltpu.sync_copy(dst_hbm.at[c, s], dst_v)
    pltpu.sync_copy(w_hbm.at[c, s], w_v)

    zero = jnp.zeros((16,), jnp.float32)

    def zb(i, carry):
        rows_v[i // 8, pl.ds((i % 8) * 16, 16)] = zero
        return carry

    lax.fori_loop(0, CH * 8, zb, 0)
    r0 = s * RPT
    for k in range(RPT // CH):
        pltpu.sync_copy(rows_v, accum.at[pl.ds(r0 + CH * k, CH), :])
    plsc.subcore_barrier()

    def chunk(j, carry):
        pltpu.async_copy(h_hbm.at[src_v.at[j]], rows_v, sem).wait()

        def grp(g, carry2):
            base = g * 16
            wrow = w_v[j, pl.ds(base, 16)]
            for e in range(16):
                wv = jnp.full((16,), wrow[e])
                for f in range(8):
                    sl = (base + e, pl.ds(16 * f, 16))
                    rows_v[sl] = rows_v[sl] * wv
            return carry2

        lax.fori_loop(0, 8, grp, 0)

        pltpu.sync_copy(rows_v, accum.at[dst_v.at[j]], add=True)
        return carry

    lax.fori_loop(0, NCH, chunk, 0)
    plsc.subcore_barrier()

    @pl.when(s < SUB - 1)
    def _():
        pltpu.sync_copy(
            accum.at[pl.ds(r0, RPT), :],
            out_hbm.at[c, pl.ds(r0, RPT), :],
        )

    @pl.when(s == SUB - 1)
    def _():
        last = N - (SUB - 1) * RPT
        pltpu.sync_copy(
            accum.at[pl.ds((SUB - 1) * RPT, last), :],
            out_hbm.at[c, pl.ds((SUB - 1) * RPT, last), :],
        )


_conv = functools.partial(
    pl.kernel,
    out_type=jax.ShapeDtypeStruct((CORES, N, D), jnp.float32),
    mesh=plsc.VectorSubcoreMesh(core_axis_name="c", subcore_axis_name="s"),
    scratch_types=[
        pltpu.VMEM((NCH, CH), jnp.int32),
        pltpu.VMEM((NCH, CH), jnp.int32),
        pltpu.VMEM((NCH, CH), jnp.float32),
        pltpu.VMEM((CH, D), jnp.float32),
        pltpu.VMEM_SHARED((NACC, D), jnp.float32),
        pltpu.SemaphoreType.DMA,
    ],
)(_conv_body)


def _prep_indices(edge_index, edge_weight):
    src = edge_index[0].astype(jnp.int32)
    dst = edge_index[1].astype(jnp.int32)
    w = edge_weight.astype(jnp.float32)
    pad = EPAD - E
    src_g = jnp.pad(src, (0, pad)).reshape(CORES, SUB, NCH, CH)
    dst_g = jnp.pad(dst, (0, pad)).reshape(CORES, SUB, NCH, CH)
    w_g = jnp.pad(w, (0, pad)).reshape(CORES, SUB, NCH, CH)
    return src_g, dst_g, w_g


def kernel(x, edge_index, edge_weight, W1, b1, W2, b2):
    src_g, dst_g, w_g = _prep_indices(edge_index, edge_weight)
    b1r = b1.reshape(1, D)
    b2r = b2.reshape(1, D)

    h = _matmul1(x, W1, b1r)
    parts = _conv(h, src_g, dst_g, w_g)
    h = _matmul2(parts, W2, b2r)
    parts = _conv(h, src_g, dst_g, w_g)
    return _add_parts(parts)
